# baseline (speedup 1.0000x reference)
import functools

import jax
import jax.numpy as jnp
from jax.experimental import pallas as pl
from jax.experimental.pallas import tpu as pltpu

_LANE = 128


@functools.lru_cache(maxsize=None)
def _make_copy(b: int, d: int, rows_per_dma: int = 8):
    nch = d // rows_per_dma

    def body(src_ref, dst_ref, sem):
        copies = [
            pltpu.make_async_copy(
                src_ref.at[pl.ds(c * rows_per_dma, rows_per_dma)],
                dst_ref.at[pl.ds(c * rows_per_dma, rows_per_dma)],
                sem,
            )
            for c in range(nch)
        ]
        for c in copies:
            c.start()
        for c in copies:
            c.wait()

    return pl.pallas_call(
        body,
        in_specs=[pl.BlockSpec(memory_space=pl.ANY)],
        out_specs=pl.BlockSpec(memory_space=pl.ANY),
        out_shape=jax.ShapeDtypeStruct((d, b), jnp.float32),
        scratch_shapes=[pltpu.SemaphoreType.DMA],
    )


@functools.lru_cache(maxsize=None)
def _make_store(b: int, n: int, d: int):
    nt = pl.cdiv(b, _LANE)
    ndb = n // _LANE
    k_steps = ndb + 3

    def _tc(k, h):
        return (h[0] // _LANE + k) % nt

    def _st(k, h):
        st = (_tc(k, h) * _LANE - h[0]) % b
        return st - jnp.where(st >= b - (_LANE - 1), b, 0)

    def _blk_a(k, h):
        return jnp.clip(_st(k, h) // _LANE, 0, ndb - 1)

    def _blk_b(k, h):
        return jnp.clip(_st(k, h) // _LANE + 1, 0, ndb - 1)

    def body(h_ref, buf_ref, da_ref, db_ref, out_ref):
        k = pl.program_id(0)
        st = _st(k, h_ref)
        s = st % _LANE
        lane = jax.lax.broadcasted_iota(jnp.int32, (d, _LANE), 1)
        iv = st + lane
        valid = (iv >= 0) & (iv < n)
        ra = pltpu.roll(da_ref[...], _LANE - s, 1)
        rb = pltpu.roll(db_ref[...], _LANE - s, 1)
        shifted = jnp.where(lane < _LANE - s, ra, rb)
        out_ref[...] = jnp.where(valid, shifted, buf_ref[...])

    grid_spec = pltpu.PrefetchScalarGridSpec(
        num_scalar_prefetch=1,
        grid=(k_steps,),
        in_specs=[
            pl.BlockSpec((d, _LANE), lambda k, h: (0, _tc(k, h))),
            pl.BlockSpec((d, _LANE), lambda k, h: (0, _blk_a(k, h))),
            pl.BlockSpec((d, _LANE), lambda k, h: (0, _blk_b(k, h))),
        ],
        out_specs=pl.BlockSpec((d, _LANE), lambda k, h: (0, _tc(k, h))),
    )

    return pl.pallas_call(
        body,
        grid_spec=grid_spec,
        out_shape=jax.ShapeDtypeStruct((d, b), jnp.float32),
        input_output_aliases={1: 0},
    )


def kernel(mem, data, head):
    n, d = data.shape
    b = mem.shape[0]
    head_arr = jnp.full((1,), head, dtype=jnp.int32) % b
    copied = _make_copy(b, d)(mem.T)
    out_t = _make_store(b, n, d)(head_arr, copied, data.T, data.T)
    return out_t.T


# speedup vs baseline: 24.4692x; 24.4692x over previous
import functools

import jax
import jax.numpy as jnp
from jax.experimental import pallas as pl
from jax.experimental.pallas import tpu as pltpu

_LANE = 128


@functools.lru_cache(maxsize=None)
def _make_copy(b: int, d: int, cb: int = 4096):
    def body(src_ref, dst_ref):
        dst_ref[...] = src_ref[...]

    return pl.pallas_call(
        body,
        grid=(pl.cdiv(b, cb),),
        in_specs=[pl.BlockSpec((d, cb), lambda k: (0, k))],
        out_specs=pl.BlockSpec((d, cb), lambda k: (0, k)),
        out_shape=jax.ShapeDtypeStruct((d, b), jnp.float32),
    )


@functools.lru_cache(maxsize=None)
def _make_store(b: int, n: int, d: int):
    nt = pl.cdiv(b, _LANE)
    ndb = n // _LANE
    k_steps = ndb + 3

    def _tc(k, h):
        return (h[0] // _LANE + k) % nt

    def _st(k, h):
        st = (_tc(k, h) * _LANE - h[0]) % b
        return st - jnp.where(st >= b - (_LANE - 1), b, 0)

    def _blk_a(k, h):
        return jnp.clip(_st(k, h) // _LANE, 0, ndb - 1)

    def _blk_b(k, h):
        return jnp.clip(_st(k, h) // _LANE + 1, 0, ndb - 1)

    def body(h_ref, buf_ref, da_ref, db_ref, out_ref):
        k = pl.program_id(0)
        st = _st(k, h_ref)
        s = st % _LANE
        lane = jax.lax.broadcasted_iota(jnp.int32, (d, _LANE), 1)
        iv = st + lane
        valid = (iv >= 0) & (iv < n)
        ra = pltpu.roll(da_ref[...], _LANE - s, 1)
        rb = pltpu.roll(db_ref[...], _LANE - s, 1)
        shifted = jnp.where(lane < _LANE - s, ra, rb)
        out_ref[...] = jnp.where(valid, shifted, buf_ref[...])

    grid_spec = pltpu.PrefetchScalarGridSpec(
        num_scalar_prefetch=1,
        grid=(k_steps,),
        in_specs=[
            pl.BlockSpec((d, _LANE), lambda k, h: (0, _tc(k, h))),
            pl.BlockSpec((d, _LANE), lambda k, h: (0, _blk_a(k, h))),
            pl.BlockSpec((d, _LANE), lambda k, h: (0, _blk_b(k, h))),
        ],
        out_specs=pl.BlockSpec((d, _LANE), lambda k, h: (0, _tc(k, h))),
    )

    return pl.pallas_call(
        body,
        grid_spec=grid_spec,
        out_shape=jax.ShapeDtypeStruct((d, b), jnp.float32),
        input_output_aliases={1: 0},
    )


def kernel(mem, data, head):
    n, d = data.shape
    b = mem.shape[0]
    head_arr = jnp.full((1,), head, dtype=jnp.int32) % b
    copied = _make_copy(b, d)(mem.T)
    out_t = _make_store(b, n, d)(head_arr, copied, data.T, data.T)
    return out_t.T


# speedup vs baseline: 28.6425x; 1.1706x over previous
import functools

import jax
import jax.numpy as jnp
from jax import lax
from jax.experimental import pallas as pl
from jax.experimental.pallas import tpu as pltpu
from jax.experimental.pallas import tpu_sc as plsc

_LANE = 128
_CW = 4608
_NB = 3


@functools.lru_cache(maxsize=None)
def _make_sc_copy(b: int, d: int):
    info = plsc.get_sparse_core_info()
    nc, ns = info.num_cores, info.num_subcores
    nw = nc * ns
    main = (b // _CW) * _CW
    ncc = main // _CW
    nrb = d // 8
    nchunks = ncc * nrb
    per_w = pl.cdiv(nchunks, nw)
    tail = b - main

    mesh = plsc.VectorSubcoreMesh(core_axis_name="c", subcore_axis_name="s")

    @functools.partial(
        pl.kernel,
        mesh=mesh,
        out_type=jax.ShapeDtypeStruct((d, b), jnp.float32),
        scratch_types=[
            pltpu.VMEM((_NB, 8, _CW), jnp.float32),
            pltpu.VMEM((8, max(tail, 8)), jnp.float32),
            pltpu.SemaphoreType.DMA((_NB,)),
            pltpu.SemaphoreType.DMA((_NB,)),
        ],
    )
    def copy(src, dst, bufs, tbuf, gsem, ssem):
        wid = lax.axis_index("s") * nc + lax.axis_index("c")

        def chunk_slices(j):
            c = wid + nw * j
            rb8 = (c // ncc) * 8
            col = (c % ncc) * _CW
            return pl.ds(rb8, 8), pl.ds(col, _CW)

        def gather(j):
            r, cl = chunk_slices(j)
            return pltpu.make_async_copy(
                src.at[r, cl], bufs.at[j % _NB], gsem.at[j % _NB]
            )

        def scatter(j):
            r, cl = chunk_slices(j)
            return pltpu.make_async_copy(
                bufs.at[j % _NB], dst.at[r, cl], ssem.at[j % _NB]
            )

        def live(j):
            return wid + nw * j < nchunks

        for j in range(min(_NB - 1, per_w)):
            @pl.when(live(j))
            def _():
                gather(j).start()

        for j in range(per_w):
            if j >= 1:
                @pl.when(live(j - 1))
                def _():
                    scatter(j - 1).wait()
            if j + _NB - 1 < per_w:
                @pl.when(live(j + _NB - 1))
                def _():
                    gather(j + _NB - 1).start()

            @pl.when(live(j))
            def _():
                gather(j).wait()
                scatter(j).start()

        @pl.when(live(per_w - 1))
        def _():
            scatter(per_w - 1).wait()

        if tail:
            @pl.when(wid < nrb)
            def _():
                pltpu.sync_copy(
                    src.at[pl.ds(wid * 8, 8), pl.ds(main, tail)], tbuf
                )
                pltpu.sync_copy(
                    tbuf, dst.at[pl.ds(wid * 8, 8), pl.ds(main, tail)]
                )

    return copy


@functools.lru_cache(maxsize=None)
def _make_store(b: int, n: int, d: int):
    nt = pl.cdiv(b, _LANE)
    ndb = n // _LANE
    k_steps = ndb + 3

    def _tc(k, h):
        return (h[0] // _LANE + k) % nt

    def _st(k, h):
        st = (_tc(k, h) * _LANE - h[0]) % b
        return st - jnp.where(st >= b - (_LANE - 1), b, 0)

    def _blk_a(k, h):
        return jnp.clip(_st(k, h) // _LANE, 0, ndb - 1)

    def _blk_b(k, h):
        return jnp.clip(_st(k, h) // _LANE + 1, 0, ndb - 1)

    def body(h_ref, buf_ref, da_ref, db_ref, out_ref):
        k = pl.program_id(0)
        st = _st(k, h_ref)
        s = st % _LANE
        lane = jax.lax.broadcasted_iota(jnp.int32, (d, _LANE), 1)
        iv = st + lane
        valid = (iv >= 0) & (iv < n)
        ra = pltpu.roll(da_ref[...], _LANE - s, 1)
        rb = pltpu.roll(db_ref[...], _LANE - s, 1)
        shifted = jnp.where(lane < _LANE - s, ra, rb)
        out_ref[...] = jnp.where(valid, shifted, buf_ref[...])

    grid_spec = pltpu.PrefetchScalarGridSpec(
        num_scalar_prefetch=1,
        grid=(k_steps,),
        in_specs=[
            pl.BlockSpec((d, _LANE), lambda k, h: (0, _tc(k, h))),
            pl.BlockSpec((d, _LANE), lambda k, h: (0, _blk_a(k, h))),
            pl.BlockSpec((d, _LANE), lambda k, h: (0, _blk_b(k, h))),
        ],
        out_specs=pl.BlockSpec((d, _LANE), lambda k, h: (0, _tc(k, h))),
    )

    return pl.pallas_call(
        body,
        grid_spec=grid_spec,
        out_shape=jax.ShapeDtypeStruct((d, b), jnp.float32),
        input_output_aliases={1: 0},
    )


def kernel(mem, data, head):
    n, d = data.shape
    b = mem.shape[0]
    head_arr = jnp.full((1,), head, dtype=jnp.int32) % b
    copied = _make_sc_copy(b, d)(mem.T)
    out_t = _make_store(b, n, d)(head_arr, copied, data.T, data.T)
    return out_t.T


# speedup vs baseline: 33.1567x; 1.1576x over previous
import functools

import jax
import jax.numpy as jnp
from jax.experimental import pallas as pl
from jax.experimental.pallas import tpu as pltpu

_LANE = 128


@functools.lru_cache(maxsize=None)
def _make_copy(b: int, d: int, cb: int = 16384):
    def body(src_ref, dst_ref):
        dst_ref[...] = src_ref[...]

    return pl.pallas_call(
        body,
        grid=(pl.cdiv(b, cb),),
        in_specs=[pl.BlockSpec((d, cb), lambda k: (0, k))],
        out_specs=pl.BlockSpec((d, cb), lambda k: (0, k)),
        out_shape=jax.ShapeDtypeStruct((d, b), jnp.float32),
    )


@functools.lru_cache(maxsize=None)
def _make_store(b: int, n: int, d: int):
    nt = pl.cdiv(b, _LANE)
    ndb = n // _LANE
    k_steps = ndb + 3

    def _tc(k, h):
        return (h[0] // _LANE + k) % nt

    def _st(k, h):
        st = (_tc(k, h) * _LANE - h[0]) % b
        return st - jnp.where(st >= b - (_LANE - 1), b, 0)

    def _blk_a(k, h):
        return jnp.clip(_st(k, h) // _LANE, 0, ndb - 1)

    def _blk_b(k, h):
        return jnp.clip(_st(k, h) // _LANE + 1, 0, ndb - 1)

    def body(h_ref, buf_ref, da_ref, db_ref, out_ref):
        k = pl.program_id(0)
        st = _st(k, h_ref)
        s = st % _LANE
        lane = jax.lax.broadcasted_iota(jnp.int32, (d, _LANE), 1)
        iv = st + lane
        valid = (iv >= 0) & (iv < n)
        ra = pltpu.roll(da_ref[...], _LANE - s, 1)
        rb = pltpu.roll(db_ref[...], _LANE - s, 1)
        shifted = jnp.where(lane < _LANE - s, ra, rb)
        out_ref[...] = jnp.where(valid, shifted, buf_ref[...])

    grid_spec = pltpu.PrefetchScalarGridSpec(
        num_scalar_prefetch=1,
        grid=(k_steps,),
        in_specs=[
            pl.BlockSpec((d, _LANE), lambda k, h: (0, _tc(k, h))),
            pl.BlockSpec((d, _LANE), lambda k, h: (0, _blk_a(k, h))),
            pl.BlockSpec((d, _LANE), lambda k, h: (0, _blk_b(k, h))),
        ],
        out_specs=pl.BlockSpec((d, _LANE), lambda k, h: (0, _tc(k, h))),
    )

    return pl.pallas_call(
        body,
        grid_spec=grid_spec,
        out_shape=jax.ShapeDtypeStruct((d, b), jnp.float32),
        input_output_aliases={1: 0},
    )


def kernel(mem, data, head):
    n, d = data.shape
    b = mem.shape[0]
    head_arr = jnp.full((1,), head, dtype=jnp.int32) % b
    copied = _make_copy(b, d)(mem.T)
    out_t = _make_store(b, n, d)(head_arr, copied, data.T, data.T)
    return out_t.T


# speedup vs baseline: 33.5945x; 1.0132x over previous
import functools

import jax
import jax.numpy as jnp
from jax.experimental import pallas as pl
from jax.experimental.pallas import tpu as pltpu

_LANE = 128


@functools.lru_cache(maxsize=None)
def _make_store(b: int, n: int, d: int):
    nt = pl.cdiv(b, _LANE)
    ndb = n // _LANE
    k_steps = ndb + 3

    def _tc(k, h):
        return (h[0] // _LANE + k) % nt

    def _st(k, h):
        st = (_tc(k, h) * _LANE - h[0]) % b
        return st - jnp.where(st >= b - (_LANE - 1), b, 0)

    def _blk_a(k, h):
        return jnp.clip(_st(k, h) // _LANE, 0, ndb - 1)

    def _blk_b(k, h):
        return jnp.clip(_st(k, h) // _LANE + 1, 0, ndb - 1)

    def body(h_ref, buf_ref, da_ref, db_ref, out_ref):
        k = pl.program_id(0)
        st = _st(k, h_ref)
        s = st % _LANE
        lane = jax.lax.broadcasted_iota(jnp.int32, (d, _LANE), 1)
        iv = st + lane
        valid = (iv >= 0) & (iv < n)
        ra = pltpu.roll(da_ref[...], _LANE - s, 1)
        rb = pltpu.roll(db_ref[...], _LANE - s, 1)
        shifted = jnp.where(lane < _LANE - s, ra, rb)
        out_ref[...] = jnp.where(valid, shifted, buf_ref[...])

    grid_spec = pltpu.PrefetchScalarGridSpec(
        num_scalar_prefetch=1,
        grid=(k_steps,),
        in_specs=[
            pl.BlockSpec((d, _LANE), lambda k, h: (0, _tc(k, h))),
            pl.BlockSpec((d, _LANE), lambda k, h: (0, _blk_a(k, h))),
            pl.BlockSpec((d, _LANE), lambda k, h: (0, _blk_b(k, h))),
        ],
        out_specs=pl.BlockSpec((d, _LANE), lambda k, h: (0, _tc(k, h))),
    )

    return pl.pallas_call(
        body,
        grid_spec=grid_spec,
        out_shape=jax.ShapeDtypeStruct((d, b), jnp.float32),
        input_output_aliases={1: 0},
    )


def kernel(mem, data, head):
    n, d = data.shape
    b = mem.shape[0]
    head_arr = jnp.full((1,), head, dtype=jnp.int32) % b
    out_t = _make_store(b, n, d)(head_arr, mem.T, data.T, data.T)
    return out_t.T
